# baseline (speedup 1.0000x reference)
import functools

import jax
import jax.numpy as jnp
from jax import lax
from jax.experimental import pallas as pl
from jax.experimental.pallas import tpu as pltpu
from jax.experimental.pallas import tpu_sc as plsc

BATCH = 16384
VOCAB = 1000000
HIDDEN = 64
NUM_SAMPLE = 5

_INFO = plsc.get_sparse_core_info()
NUM_WORKERS = _INFO.num_cores * _INFO.num_subcores
ROWS_PER_WORKER = BATCH // NUM_WORKERS
CHUNK_B = 16
NUM_CHUNKS = ROWS_PER_WORKER // CHUNK_B
CHUNK_ROWS = CHUNK_B * NUM_SAMPLE
HVECS = HIDDEN // 16


def _make_sc_kernel():
  mesh = plsc.VectorSubcoreMesh(core_axis_name="c", subcore_axis_name="s")

  @functools.partial(
      pl.kernel,
      mesh=mesh,
      out_type=jax.ShapeDtypeStruct((BATCH * NUM_SAMPLE,), jnp.float32),
      compiler_params=pltpu.CompilerParams(
          needs_layout_passes=False, use_tc_tiling_on_sc=False),
      scratch_types=[
          pltpu.VMEM((CHUNK_ROWS,), jnp.int32),
          pltpu.VMEM((CHUNK_ROWS * HIDDEN,), jnp.float32),
          pltpu.VMEM((CHUNK_B, HIDDEN), jnp.float32),
          pltpu.VMEM((CHUNK_ROWS,), jnp.float32),
          pltpu.SemaphoreType.DMA,
      ],
  )
  def neg_sampling(inputs_hbm, idx_hbm, table_hbm, out_hbm,
                   idx_v, rows_v, inp_v, out_v, sem):
    wid = lax.axis_index("s") * _INFO.num_cores + lax.axis_index("c")
    iota = lax.iota(jnp.int32, 16)
    perms = {w: lax.bitwise_xor(iota, w) for w in (1, 2, 4, 8)}
    masks = {w: lax.bitwise_and(iota, w) == 0 for w in (1, 2, 4, 8)}

    def chunk_body(c, carry):
      base_b = wid * ROWS_PER_WORKER + c * CHUNK_B
      out_base = base_b * NUM_SAMPLE
      pltpu.sync_copy(idx_hbm.at[pl.ds(out_base, CHUNK_ROWS)], idx_v)
      pltpu.sync_copy(inputs_hbm.at[pl.ds(base_b, CHUNK_B)], inp_v)
      for g in range(CHUNK_ROWS // 16):
        rv = idx_v[pl.ds(g * 16, 16)] * HIDDEN
        for t in range(16):
          k = g * 16 + t
          pltpu.async_copy(table_hbm.at[pl.ds(pl.multiple_of(rv[t], 8), HIDDEN)],
                           rows_v.at[pl.ds(k * HIDDEN, HIDDEN)], sem)
      pltpu.make_async_copy(
          table_hbm.at[pl.ds(0, CHUNK_ROWS * HIDDEN)], rows_v, sem).wait()

      ivecs = [[inp_v[b, pl.ds(j * 16, 16)] for j in range(HVECS)]
               for b in range(CHUNK_B)]
      for g in range(CHUNK_ROWS // 16):
        cur = []
        for t in range(16):
          k = g * 16 + t
          iv = ivecs[k // NUM_SAMPLE]
          acc = iv[0] * rows_v[pl.ds(k * HIDDEN, 16)]
          for j in range(1, HVECS):
            acc = acc + iv[j] * rows_v[pl.ds(k * HIDDEN + j * 16, 16)]
          cur.append(acc)
        for w in (1, 2, 4, 8):
          nxt = []
          for i in range(0, len(cur), 2):
            a, b = cur[i], cur[i + 1]
            t1 = jnp.where(masks[w], a, b)
            t2 = jnp.where(masks[w], b, a)
            nxt.append(t1 + t2.at[perms[w]].get(mode="promise_in_bounds"))
          cur = nxt
        out_v[pl.ds(g * 16, 16)] = 1.0 / (1.0 + jnp.exp(-cur[0]))

      pltpu.sync_copy(out_v, out_hbm.at[pl.ds(out_base, CHUNK_ROWS)])
      return carry

    lax.fori_loop(0, NUM_CHUNKS, chunk_body, 0)

  return neg_sampling


_sc_kernel = _make_sc_kernel()


def kernel(inputs, idxs, out_embedding):
  idx_flat = idxs.astype(jnp.int32).reshape(BATCH * NUM_SAMPLE)
  table_flat = out_embedding.reshape(VOCAB * HIDDEN)
  out_flat = _sc_kernel(inputs, idx_flat, table_flat)
  return out_flat.reshape(BATCH, NUM_SAMPLE)


# speedup vs baseline: 1.4987x; 1.4987x over previous
import functools

import jax
import jax.numpy as jnp
from jax import lax
from jax.experimental import pallas as pl
from jax.experimental.pallas import tpu as pltpu
from jax.experimental.pallas import tpu_sc as plsc

BATCH = 16384
VOCAB = 1000000
HIDDEN = 64
NUM_SAMPLE = 5

_INFO = plsc.get_sparse_core_info()
NUM_WORKERS = _INFO.num_cores * _INFO.num_subcores
ROWS_PER_WORKER = BATCH // NUM_WORKERS
CHUNK_B = 16
NUM_CHUNKS = ROWS_PER_WORKER // CHUNK_B
CHUNK_ROWS = CHUNK_B * NUM_SAMPLE
HVECS = HIDDEN // 16


def _make_sc_kernel():
  mesh = plsc.VectorSubcoreMesh(core_axis_name="c", subcore_axis_name="s")

  @functools.partial(
      pl.kernel,
      mesh=mesh,
      out_type=jax.ShapeDtypeStruct((BATCH * NUM_SAMPLE,), jnp.float32),
      compiler_params=pltpu.CompilerParams(
          needs_layout_passes=True, use_tc_tiling_on_sc=True),
      scratch_types=[
          pltpu.VMEM((CHUNK_ROWS,), jnp.int32),
          pltpu.VMEM((CHUNK_ROWS, HIDDEN), jnp.float32),
          pltpu.VMEM((CHUNK_B, HIDDEN), jnp.float32),
          pltpu.VMEM((CHUNK_ROWS,), jnp.float32),
          pltpu.SemaphoreType.DMA,
      ],
  )
  def neg_sampling(inputs_hbm, idx_hbm, table_hbm, out_hbm,
                   idx_v, rows_v, inp_v, out_v, sem):
    wid = lax.axis_index("s") * _INFO.num_cores + lax.axis_index("c")
    iota = lax.iota(jnp.int32, 16)
    perms = {w: lax.bitwise_xor(iota, w) for w in (1, 2, 4, 8)}
    masks = {w: lax.bitwise_and(iota, w) == 0 for w in (1, 2, 4, 8)}

    def chunk_body(c, carry):
      base_b = wid * ROWS_PER_WORKER + c * CHUNK_B
      out_base = base_b * NUM_SAMPLE
      pltpu.sync_copy(idx_hbm.at[pl.ds(out_base, CHUNK_ROWS)], idx_v)
      pltpu.sync_copy(inputs_hbm.at[pl.ds(base_b, CHUNK_B)], inp_v)
      for g in range(CHUNK_ROWS // 16):
        rv = idx_v[pl.ds(g * 16, 16)]
        for t in range(16):
          k = g * 16 + t
          pltpu.async_copy(table_hbm.at[pl.ds(rv[t], 1)],
                           rows_v.at[pl.ds(k, 1)], sem)
      pltpu.make_async_copy(
          table_hbm.at[pl.ds(0, CHUNK_ROWS)], rows_v, sem).wait()

      ivecs = [[inp_v[b, pl.ds(j * 16, 16)] for j in range(HVECS)]
               for b in range(CHUNK_B)]
      for g in range(CHUNK_ROWS // 16):
        cur = []
        for t in range(16):
          k = g * 16 + t
          iv = ivecs[k // NUM_SAMPLE]
          acc = iv[0] * rows_v[k, pl.ds(0, 16)]
          for j in range(1, HVECS):
            acc = acc + iv[j] * rows_v[k, pl.ds(j * 16, 16)]
          cur.append(acc)
        for w in (1, 2, 4, 8):
          nxt = []
          for i in range(0, len(cur), 2):
            a, b = cur[i], cur[i + 1]
            t1 = jnp.where(masks[w], a, b)
            t2 = jnp.where(masks[w], b, a)
            nxt.append(t1 + t2.at[perms[w]].get(mode="promise_in_bounds"))
          cur = nxt
        out_v[pl.ds(g * 16, 16)] = 1.0 / (1.0 + jnp.exp(-cur[0]))

      pltpu.sync_copy(out_v, out_hbm.at[pl.ds(out_base, CHUNK_ROWS)])
      return carry

    lax.fori_loop(0, NUM_CHUNKS, chunk_body, 0)

  return neg_sampling


_sc_kernel = _make_sc_kernel()


def kernel(inputs, idxs, out_embedding):
  idx_flat = idxs.astype(jnp.int32).reshape(BATCH * NUM_SAMPLE)
  out_flat = _sc_kernel(inputs, idx_flat, out_embedding)
  return out_flat.reshape(BATCH, NUM_SAMPLE)
